# Initial kernel scaffold; baseline (speedup 1.0000x reference)
import jax
import jax.numpy as jnp
from jax.experimental import pallas as pl


def kernel(word, pos1, pos2, word_table, pos1_table, pos2_table):
    raise NotImplementedError("write your pallas kernel here")


# speedup vs baseline: 3.4331x; 3.4331x over previous
import functools

import jax
import jax.numpy as jnp
from jax import lax
from jax.experimental import pallas as pl
from jax.experimental.pallas import tpu as pltpu
from jax.experimental.pallas import tpu_sc as plsc

B, L = 4096, 200
N = B * L
WDIM, PDIM = 50, 5
ODIM = WDIM + 2 * PDIM
GDIM = 64
PTAB_HALF = 2 * 200 * PDIM
NC, NS = 2, 16
NW = NC * NS
ROWS_PER_W = N // NW
CHUNK = 512
NCHUNK = ROWS_PER_W // CHUNK
SUB = 128
NSUB = CHUNK // SUB
GROUPS = CHUNK // 16


def _body(word2d, p1f, p2f, wtab, ptab_h, out,
          widx, p1i, p2i, ptab_v, gbuf, obuf, sem):
    wid = lax.axis_index("s") * NC + lax.axis_index("c")
    pltpu.sync_copy(ptab_h, ptab_v)
    iota16 = lax.iota(jnp.int32, 16)

    def chunk_body(k, carry):
        base = pl.multiple_of(wid * ROWS_PER_W + k * CHUNK, CHUNK)
        rowb = pl.multiple_of(wid * (ROWS_PER_W // SUB) + k * NSUB, NSUB)
        pltpu.sync_copy(word2d.at[pl.ds(rowb, NSUB)], widx)
        pltpu.sync_copy(p1f.at[pl.ds(base, CHUNK)], p1i)
        pltpu.sync_copy(p2f.at[pl.ds(base, CHUNK)], p2i)
        cps = [pltpu.async_copy(wtab.at[widx.at[j]],
                                gbuf.at[pl.ds(j * SUB, SUB)], sem)
               for j in range(NSUB)]
        for cp in cps:
            cp.wait()

        def gbody(g, c2):
            r0 = pl.multiple_of(g * 16, 16)
            rows = g * 16 + iota16
            for c in range(WDIM):
                cc = jnp.full((16,), c, jnp.int32)
                v = plsc.load_gather(gbuf, [rows, cc])
                plsc.store_scatter(obuf, [rows, cc], v)
            pv1 = p1i[pl.ds(r0, 16)]
            pv2 = p2i[pl.ds(r0, 16)]
            for c in range(PDIM):
                v1 = plsc.load_gather(ptab_v, [pv1 * PDIM + c])
                plsc.store_scatter(
                    obuf, [rows, jnp.full((16,), WDIM + c, jnp.int32)], v1)
                v2 = plsc.load_gather(ptab_v, [PTAB_HALF + pv2 * PDIM + c])
                plsc.store_scatter(
                    obuf, [rows, jnp.full((16,), WDIM + PDIM + c, jnp.int32)],
                    v2)
            return c2

        lax.fori_loop(0, GROUPS, gbody, 0)
        pltpu.sync_copy(obuf, out.at[pl.ds(base, CHUNK)])
        return carry

    lax.fori_loop(0, NCHUNK, chunk_body, 0)


_sc_lookup = functools.partial(
    pl.kernel,
    out_type=jax.ShapeDtypeStruct((N, ODIM), jnp.float32),
    mesh=plsc.VectorSubcoreMesh(core_axis_name="c", subcore_axis_name="s"),
    compiler_params=pltpu.CompilerParams(
        needs_layout_passes=False, use_tc_tiling_on_sc=False),
    scratch_types=[
        pltpu.VMEM((NSUB, SUB), jnp.int32),
        pltpu.VMEM((CHUNK,), jnp.int32),
        pltpu.VMEM((CHUNK,), jnp.int32),
        pltpu.VMEM((2 * PTAB_HALF,), jnp.float32),
        pltpu.VMEM((CHUNK, GDIM), jnp.float32),
        pltpu.VMEM((CHUNK, ODIM), jnp.float32),
        pltpu.SemaphoreType.DMA,
    ],
)(_body)


@jax.jit
def _run(word, pos1, pos2, word_table, pos1_table, pos2_table):
    w = word.reshape(N // SUB, SUB).astype(jnp.int32)
    p1 = pos1.reshape(N).astype(jnp.int32)
    p2 = pos2.reshape(N).astype(jnp.int32)
    wtab = jnp.pad(word_table, ((0, 0), (0, GDIM - WDIM)))
    ptab = jnp.concatenate(
        [pos1_table.reshape(-1), pos2_table.reshape(-1)])
    out = _sc_lookup(w, p1, p2, wtab, ptab)
    return out.reshape(B, L, ODIM)


def kernel(word, pos1, pos2, word_table, pos1_table, pos2_table):
    return _run(word, pos1, pos2, word_table, pos1_table, pos2_table)


# speedup vs baseline: 7.2733x; 2.1186x over previous
import functools

import jax
import jax.numpy as jnp
from jax import lax
from jax.experimental import pallas as pl
from jax.experimental.pallas import tpu as pltpu
from jax.experimental.pallas import tpu_sc as plsc

B, L = 4096, 200
N = B * L
WDIM, PDIM = 50, 5
ODIM = WDIM + 2 * PDIM
GDIM = 64
PTAB_HALF = 2 * 200 * PDIM
NC, NS = 2, 16
NW = NC * NS
ROWS_PER_W = N // NW
CHUNK = 512
NCHUNK = ROWS_PER_W // CHUNK
SUB = 128
NSUB = CHUNK // SUB
GROUPS = CHUNK // 16


def _body(word2d, p1f, p2f, wtab, ptab_h, out,
          widx, p1i, p2i, ptab_v, gbuf, obuf, sem):
    wid = lax.axis_index("s") * NC + lax.axis_index("c")
    pltpu.sync_copy(ptab_h, ptab_v)
    iota16 = lax.iota(jnp.int32, 16)

    def chunk_body(k, carry):
        base = pl.multiple_of(wid * ROWS_PER_W + k * CHUNK, CHUNK)
        rowb = pl.multiple_of(wid * (ROWS_PER_W // SUB) + k * NSUB, NSUB)
        pltpu.sync_copy(word2d.at[pl.ds(rowb, NSUB)], widx)
        pltpu.sync_copy(p1f.at[pl.ds(base, CHUNK)], p1i)
        pltpu.sync_copy(p2f.at[pl.ds(base, CHUNK)], p2i)
        cps = [pltpu.async_copy(wtab.at[widx.at[j]],
                                gbuf.at[pl.ds(j * SUB, SUB)], sem)
               for j in range(NSUB)]
        for cp in cps:
            cp.wait()

        def gbody(g, c2):
            r0 = pl.multiple_of(g * 16, 16)
            rows = g * 16 + iota16
            pv1 = p1i[pl.ds(r0, 16)]
            pv2 = p2i[pl.ds(r0, 16)]
            for c in range(PDIM):
                v1 = plsc.load_gather(ptab_v, [pv1 * PDIM + c])
                plsc.store_scatter(
                    gbuf, [rows, jnp.full((16,), WDIM + c, jnp.int32)], v1)
                v2 = plsc.load_gather(ptab_v, [PTAB_HALF + pv2 * PDIM + c])
                if c == 0:
                    plsc.store_scatter(
                        gbuf, [rows, jnp.full((16,), WDIM + PDIM, jnp.int32)],
                        v2)
                else:
                    plsc.store_scatter(
                        obuf,
                        [rows, jnp.full((16,), WDIM + PDIM + c, jnp.int32)],
                        v2)
            return c2

        lax.fori_loop(0, GROUPS, gbody, 0)
        pltpu.sync_copy(obuf, out.at[pl.ds(base, CHUNK)])
        pltpu.sync_copy(gbuf.at[:, pl.ds(0, 56)],
                        out.at[pl.ds(base, CHUNK), pl.ds(0, 56)])
        return carry

    lax.fori_loop(0, NCHUNK, chunk_body, 0)


_sc_lookup = functools.partial(
    pl.kernel,
    out_type=jax.ShapeDtypeStruct((N, ODIM), jnp.float32),
    mesh=plsc.VectorSubcoreMesh(core_axis_name="c", subcore_axis_name="s"),
    compiler_params=pltpu.CompilerParams(
        needs_layout_passes=False, use_tc_tiling_on_sc=False),
    scratch_types=[
        pltpu.VMEM((NSUB, SUB), jnp.int32),
        pltpu.VMEM((CHUNK,), jnp.int32),
        pltpu.VMEM((CHUNK,), jnp.int32),
        pltpu.VMEM((2 * PTAB_HALF,), jnp.float32),
        pltpu.VMEM((CHUNK, GDIM), jnp.float32),
        pltpu.VMEM((CHUNK, ODIM), jnp.float32),
        pltpu.SemaphoreType.DMA,
    ],
)(_body)


@jax.jit
def _run(word, pos1, pos2, word_table, pos1_table, pos2_table):
    w = word.reshape(N // SUB, SUB).astype(jnp.int32)
    p1 = pos1.reshape(N).astype(jnp.int32)
    p2 = pos2.reshape(N).astype(jnp.int32)
    wtab = jnp.pad(word_table, ((0, 0), (0, GDIM - WDIM)))
    ptab = jnp.concatenate(
        [pos1_table.reshape(-1), pos2_table.reshape(-1)])
    out = _sc_lookup(w, p1, p2, wtab, ptab)
    return out.reshape(B, L, ODIM)


def kernel(word, pos1, pos2, word_table, pos1_table, pos2_table):
    return _run(word, pos1, pos2, word_table, pos1_table, pos2_table)


# speedup vs baseline: 8.0946x; 1.1129x over previous
import functools

import jax
import jax.numpy as jnp
from jax import lax
from jax.experimental import pallas as pl
from jax.experimental.pallas import tpu as pltpu
from jax.experimental.pallas import tpu_sc as plsc

B, L = 4096, 200
N = B * L
WDIM, PDIM = 50, 5
ODIM = WDIM + 2 * PDIM
GDIM = 56
PTAB_HALF = 2 * 200 * PDIM
NC, NS = 2, 16
NW = NC * NS
ROWS_PER_W = N // NW
CHUNK = 512
NCHUNK = ROWS_PER_W // CHUNK
SUB = 128
NSUB = CHUNK // SUB
GROUPS = CHUNK // 16


def _body(word2d, p1f, p2f, wtab, ptab_h, out,
          widx, p1i, p2i, ptab_v, gbuf, obuf, sem):
    wid = lax.axis_index("s") * NC + lax.axis_index("c")
    pltpu.sync_copy(ptab_h, ptab_v)
    iota16 = lax.iota(jnp.int32, 16)

    def chunk_body(k, carry):
        base = pl.multiple_of(wid * ROWS_PER_W + k * CHUNK, CHUNK)
        rowb = pl.multiple_of(wid * (ROWS_PER_W // SUB) + k * NSUB, NSUB)
        pltpu.sync_copy(word2d.at[pl.ds(rowb, NSUB)], widx)
        pltpu.sync_copy(p1f.at[pl.ds(base, CHUNK)], p1i)
        pltpu.sync_copy(p2f.at[pl.ds(base, CHUNK)], p2i)
        cps = [pltpu.async_copy(wtab.at[widx.at[j]],
                                gbuf.at[pl.ds(j * SUB, SUB)], sem)
               for j in range(NSUB)]
        for cp in cps:
            cp.wait()

        def gbody(g, c2):
            r0 = pl.multiple_of(g * 16, 16)
            rows = g * 16 + iota16
            pv1 = p1i[pl.ds(r0, 16)]
            pv2 = p2i[pl.ds(r0, 16)]
            for c in range(PDIM):
                v1 = plsc.load_gather(ptab_v, [pv1 * PDIM + c])
                plsc.store_scatter(
                    gbuf, [rows, jnp.full((16,), WDIM + c, jnp.int32)], v1)
                v2 = plsc.load_gather(ptab_v, [PTAB_HALF + pv2 * PDIM + c])
                if c == 0:
                    plsc.store_scatter(
                        gbuf, [rows, jnp.full((16,), WDIM + PDIM, jnp.int32)],
                        v2)
                else:
                    plsc.store_scatter(
                        obuf,
                        [rows, jnp.full((16,), WDIM + PDIM + c, jnp.int32)],
                        v2)
            return c2

        lax.fori_loop(0, GROUPS, gbody, 0)
        pltpu.sync_copy(obuf, out.at[pl.ds(base, CHUNK)])
        pltpu.sync_copy(gbuf.at[:, pl.ds(0, 56)],
                        out.at[pl.ds(base, CHUNK), pl.ds(0, 56)])
        return carry

    lax.fori_loop(0, NCHUNK, chunk_body, 0)


_sc_lookup = functools.partial(
    pl.kernel,
    out_type=jax.ShapeDtypeStruct((N, ODIM), jnp.float32),
    mesh=plsc.VectorSubcoreMesh(core_axis_name="c", subcore_axis_name="s"),
    compiler_params=pltpu.CompilerParams(
        needs_layout_passes=False, use_tc_tiling_on_sc=False),
    scratch_types=[
        pltpu.VMEM((NSUB, SUB), jnp.int32),
        pltpu.VMEM((CHUNK,), jnp.int32),
        pltpu.VMEM((CHUNK,), jnp.int32),
        pltpu.VMEM((2 * PTAB_HALF,), jnp.float32),
        pltpu.VMEM((CHUNK, GDIM), jnp.float32),
        pltpu.VMEM((CHUNK, ODIM), jnp.float32),
        pltpu.SemaphoreType.DMA,
    ],
)(_body)


@jax.jit
def _run(word, pos1, pos2, word_table, pos1_table, pos2_table):
    w = word.reshape(N // SUB, SUB).astype(jnp.int32)
    p1 = pos1.reshape(N).astype(jnp.int32)
    p2 = pos2.reshape(N).astype(jnp.int32)
    wtab = jnp.pad(word_table, ((0, 0), (0, GDIM - WDIM)))
    ptab = jnp.concatenate(
        [pos1_table.reshape(-1), pos2_table.reshape(-1)])
    out = _sc_lookup(w, p1, p2, wtab, ptab)
    return out.reshape(B, L, ODIM)


def kernel(word, pos1, pos2, word_table, pos1_table, pos2_table):
    return _run(word, pos1, pos2, word_table, pos1_table, pos2_table)


# speedup vs baseline: 9.7970x; 1.2103x over previous
import functools

import jax
import jax.numpy as jnp
from jax import lax
from jax.experimental import pallas as pl
from jax.experimental.pallas import tpu as pltpu
from jax.experimental.pallas import tpu_sc as plsc

B, L = 4096, 200
N = B * L
WDIM, PDIM = 50, 5
ODIM = WDIM + 2 * PDIM
GDIM = 56
PTAB_HALF = 2 * 200 * PDIM
NC, NS = 2, 16
NW = NC * NS
ROWS_PER_W = N // NW
CHUNK = 256
NCHUNK = ROWS_PER_W // CHUNK
SUB = 128
NSUB = CHUNK // SUB
GROUPS = CHUNK // 16


def _body(word2d, p1f, p2f, wtab, ptab_h, out,
          widx, p1i, p2i, ptab_v, gbuf, obuf, sem_i, sem_g, sem_w):
    wid = lax.axis_index("s") * NC + lax.axis_index("c")
    pltpu.sync_copy(ptab_h, ptab_v)
    iota16 = lax.iota(jnp.int32, 16)

    def bases(k):
        base = pl.multiple_of(wid * ROWS_PER_W + k * CHUNK, CHUNK)
        rowb = pl.multiple_of(wid * (ROWS_PER_W // SUB) + k * NSUB, NSUB)
        return base, rowb

    def fire_idx(k, s):
        base, rowb = bases(k)
        pltpu.async_copy(word2d.at[pl.ds(rowb, NSUB)], widx.at[s], sem_i)
        pltpu.async_copy(p1f.at[pl.ds(base, CHUNK)], p1i.at[s], sem_i)
        pltpu.async_copy(p2f.at[pl.ds(base, CHUNK)], p2i.at[s], sem_i)

    def wait_idx(s):
        pltpu.make_async_copy(
            word2d.at[pl.ds(0, NSUB)], widx.at[s], sem_i).wait()
        pltpu.make_async_copy(p1f.at[pl.ds(0, CHUNK)], p1i.at[s], sem_i).wait()
        pltpu.make_async_copy(p2f.at[pl.ds(0, CHUNK)], p2i.at[s], sem_i).wait()

    def fire_gathers(s):
        for j in range(NSUB):
            pltpu.async_copy(wtab.at[widx.at[s].at[j]],
                             gbuf.at[s].at[pl.ds(j * SUB, SUB)], sem_g)

    def wait_gathers(s):
        for j in range(NSUB):
            pltpu.make_async_copy(
                wtab.at[pl.ds(0, SUB)],
                gbuf.at[s].at[pl.ds(j * SUB, SUB)], sem_g).wait()

    def fill(s):
        def gbody(g, c2):
            r0 = pl.multiple_of(g * 16, 16)
            rows = g * 16 + iota16
            pv1 = p1i.at[s][pl.ds(r0, 16)]
            pv2 = p2i.at[s][pl.ds(r0, 16)]
            for c in range(PDIM):
                v1 = plsc.load_gather(ptab_v, [pv1 * PDIM + c])
                plsc.store_scatter(
                    gbuf.at[s],
                    [rows, jnp.full((16,), WDIM + c, jnp.int32)], v1)
                v2 = plsc.load_gather(ptab_v, [PTAB_HALF + pv2 * PDIM + c])
                if c == 0:
                    plsc.store_scatter(
                        gbuf.at[s],
                        [rows, jnp.full((16,), WDIM + PDIM, jnp.int32)], v2)
                else:
                    plsc.store_scatter(
                        obuf.at[s],
                        [rows, jnp.full((16,), WDIM + PDIM + c, jnp.int32)],
                        v2)
            return c2

        lax.fori_loop(0, GROUPS, gbody, 0)

    def fire_writes(k, s):
        base, _ = bases(k)
        pltpu.async_copy(obuf.at[s], out.at[pl.ds(base, CHUNK)], sem_w)
        pltpu.async_copy(gbuf.at[s, :, pl.ds(0, GDIM)],
                         out.at[pl.ds(base, CHUNK), pl.ds(0, GDIM)], sem_w)

    def wait_writes():
        pltpu.make_async_copy(
            obuf.at[0], out.at[pl.ds(0, CHUNK)], sem_w).wait()
        pltpu.make_async_copy(
            gbuf.at[0, :, pl.ds(0, GDIM)],
            out.at[pl.ds(0, CHUNK), pl.ds(0, GDIM)], sem_w).wait()

    def phase(kk, s, first, last, pre_idx=True):
        wait_gathers(s)
        if not last:
            wait_idx(1 - s)
        if not first:
            wait_writes()
        if not last:
            fire_gathers(1 - s)
        fill(s)
        if not last and pre_idx:
            fire_idx(kk + 2, s)
        fire_writes(kk, s)

    base0, rowb0 = bases(0)
    pltpu.sync_copy(word2d.at[pl.ds(rowb0, NSUB)], widx.at[0])
    pltpu.sync_copy(p1f.at[pl.ds(base0, CHUNK)], p1i.at[0])
    pltpu.sync_copy(p2f.at[pl.ds(base0, CHUNK)], p2i.at[0])
    fire_gathers(0)
    fire_idx(1, 1)
    phase(0, 0, first=True, last=False)
    phase(1, 1, first=False, last=False)

    def loop_body(i, carry):
        kk = 2 * i
        phase(kk, 0, first=False, last=False)
        phase(kk + 1, 1, first=False, last=False)
        return carry

    lax.fori_loop(1, NCHUNK // 2 - 1, loop_body, 0)

    phase(NCHUNK - 2, 0, first=False, last=False, pre_idx=False)
    phase(NCHUNK - 1, 1, first=False, last=True)
    wait_writes()


_sc_lookup = functools.partial(
    pl.kernel,
    out_type=jax.ShapeDtypeStruct((N, ODIM), jnp.float32),
    mesh=plsc.VectorSubcoreMesh(core_axis_name="c", subcore_axis_name="s"),
    compiler_params=pltpu.CompilerParams(
        needs_layout_passes=False, use_tc_tiling_on_sc=False),
    scratch_types=[
        pltpu.VMEM((2, NSUB, SUB), jnp.int32),
        pltpu.VMEM((2, CHUNK), jnp.int32),
        pltpu.VMEM((2, CHUNK), jnp.int32),
        pltpu.VMEM((2 * PTAB_HALF,), jnp.float32),
        pltpu.VMEM((2, CHUNK, GDIM), jnp.float32),
        pltpu.VMEM((2, CHUNK, ODIM), jnp.float32),
        pltpu.SemaphoreType.DMA,
        pltpu.SemaphoreType.DMA,
        pltpu.SemaphoreType.DMA,
    ],
)(_body)


@jax.jit
def _run(word, pos1, pos2, word_table, pos1_table, pos2_table):
    w = word.reshape(N // SUB, SUB).astype(jnp.int32)
    p1 = pos1.reshape(N).astype(jnp.int32)
    p2 = pos2.reshape(N).astype(jnp.int32)
    wtab = jnp.pad(word_table, ((0, 0), (0, GDIM - WDIM)))
    ptab = jnp.concatenate(
        [pos1_table.reshape(-1), pos2_table.reshape(-1)])
    out = _sc_lookup(w, p1, p2, wtab, ptab)
    return out.reshape(B, L, ODIM)


def kernel(word, pos1, pos2, word_table, pos1_table, pos2_table):
    return _run(word, pos1, pos2, word_table, pos1_table, pos2_table)
